# Initial kernel scaffold; baseline (speedup 1.0000x reference)
import jax
import jax.numpy as jnp
from jax.experimental import pallas as pl


def kernel(sub_tokens_indices, embeddings_weight):
    raise NotImplementedError("write your pallas kernel here")


# speedup vs baseline: 2.5232x; 2.5232x over previous
import functools

import jax
import jax.numpy as jnp
from jax import lax
from jax.experimental import pallas as pl
from jax.experimental.pallas import tpu as pltpu
from jax.experimental.pallas import tpu_sc as plsc

_BATCH = 16384
_SUBLEN = 50
_EMBED = 64
_NC = 2
_NS = 16
_NW = _NC * _NS
_ROWS_PER_W = _BATCH // _NW
_CHUNK = 16
_NCHUNKS = _ROWS_PER_W // _CHUNK
_IDX_COLS = 100
_GATHERS = _CHUNK * _SUBLEN // _IDX_COLS


def _sc_body(idx2d_hbm, table_hbm, out_hbm, idx_v, rows_v, out_v, sem):
    wid = lax.axis_index("s") * _NC + lax.axis_index("c")
    row0 = wid * _ROWS_PER_W

    lanes = lax.iota(jnp.int32, 16)
    g_row = lanes >> 1
    g_col0 = (lanes & 1) * _SUBLEN

    def chunk_body(c, carry):
        base = pl.multiple_of(row0 + c * _CHUNK, _CHUNK)
        pltpu.sync_copy(
            idx2d_hbm.at[pl.ds(pl.multiple_of(base // 2, 8), _CHUNK // 2)],
            idx_v)
        copies = []
        for j in range(_GATHERS):
            copies.append(pltpu.async_copy(
                table_hbm.at[idx_v.at[j]],
                rows_v.at[pl.ds(j * _IDX_COLS, _IDX_COLS)],
                sem))
        cnt = jnp.zeros((16,), jnp.float32)
        for s in range(_SUBLEN):
            v = plsc.load_gather(idx_v, [g_row, g_col0 + s])
            cnt = cnt + jnp.where(v != 0, 1.0, 0.0)
        rcp = 1.0 / cnt
        for cp in copies:
            cp.wait()
        for b in range(_CHUNK):
            r = rcp[b]

            def sbody(s, accs, b=b):
                row = b * _SUBLEN + s
                return tuple(
                    accs[d] + rows_v[row, pl.ds(d * 16, 16)] for d in range(4)
                )

            z = jnp.zeros((16,), jnp.float32)
            accs = lax.fori_loop(0, _SUBLEN, sbody, (z, z, z, z), unroll=5)
            for d in range(4):
                out_v[b, pl.ds(d * 16, 16)] = accs[d] * r
        pltpu.sync_copy(out_v, out_hbm.at[pl.ds(base, _CHUNK)])
        return carry

    lax.fori_loop(0, _NCHUNKS, chunk_body, 0)


@jax.jit
def _sub_token_embed(idx2d, table):
    mesh = plsc.VectorSubcoreMesh(core_axis_name="c", subcore_axis_name="s")
    return pl.kernel(
        _sc_body,
        out_type=jax.ShapeDtypeStruct((_BATCH, _EMBED), jnp.float32),
        mesh=mesh,
        scratch_types=[
            pltpu.VMEM((_CHUNK // 2, _IDX_COLS), jnp.int32),
            pltpu.VMEM((_CHUNK * _SUBLEN, _EMBED), jnp.float32),
            pltpu.VMEM((_CHUNK, _EMBED), jnp.float32),
            pltpu.SemaphoreType.DMA,
        ],
        compiler_params=pltpu.CompilerParams(
            use_tc_tiling_on_sc=False, needs_layout_passes=False),
    )(idx2d, table)


def kernel(sub_tokens_indices, embeddings_weight):
    idx2d = sub_tokens_indices.astype(jnp.int32).reshape(
        _BATCH // 2, 2 * _SUBLEN)
    return _sub_token_embed(idx2d, embeddings_weight)


# speedup vs baseline: 2.7650x; 1.0958x over previous
import functools

import jax
import jax.numpy as jnp
from jax import lax
from jax.experimental import pallas as pl
from jax.experimental.pallas import tpu as pltpu
from jax.experimental.pallas import tpu_sc as plsc

_BATCH = 16384
_SUBLEN = 50
_EMBED = 64
_NC = 2
_NS = 16
_NW = _NC * _NS
_ROWS_PER_W = _BATCH // _NW
_CHUNK = 16
_NCHUNKS = _ROWS_PER_W // _CHUNK
_IDX_COLS = 100
_GATHERS = _CHUNK * _SUBLEN // _IDX_COLS


def _sc_body(idx2d_hbm, table_hbm, out_hbm,
             idx_a, idx_b, rows_a, rows_b, out_v, sem_i, sem_g):
    wid = lax.axis_index("s") * _NC + lax.axis_index("c")
    row0 = wid * _ROWS_PER_W
    idx_bufs = (idx_a, idx_b)
    rows_bufs = (rows_a, rows_b)

    lanes = lax.iota(jnp.int32, 16)
    g_row = lanes >> 1
    g_col0 = (lanes & 1) * _SUBLEN

    def idx_src(chunk):
        base = pl.multiple_of(row0 + chunk * _CHUNK, _CHUNK)
        return idx2d_hbm.at[pl.ds(pl.multiple_of(base // 2, 8), _CHUNK // 2)]

    def issue_idx(chunk, p):
        pltpu.async_copy(idx_src(chunk), idx_bufs[p], sem_i)

    def wait_idx(chunk, p):
        pltpu.make_async_copy(idx_src(chunk), idx_bufs[p], sem_i).wait()

    def gather_copies(p):
        return [
            pltpu.make_async_copy(
                table_hbm.at[idx_bufs[p].at[j]],
                rows_bufs[p].at[pl.ds(j * _IDX_COLS, _IDX_COLS)],
                sem_g)
            for j in range(_GATHERS)
        ]

    def issue_gathers(p):
        for cp in gather_copies(p):
            cp.start()

    def wait_gathers(p):
        for cp in gather_copies(p):
            cp.wait()

    def count_rcp(p):
        cnt = jnp.zeros((16,), jnp.float32)
        for s in range(_SUBLEN):
            v = plsc.load_gather(idx_bufs[p], [g_row, g_col0 + s])
            cnt = cnt + jnp.where(v != 0, 1.0, 0.0)
        return 1.0 / cnt

    def compute(chunk, p, rcp):
        rows_v = rows_bufs[p]
        for b in range(_CHUNK):
            r = rcp[b]

            def sbody(s, accs, b=b):
                row = b * _SUBLEN + s
                return tuple(
                    accs[d] + rows_v[row, pl.ds(d * 16, 16)] for d in range(4)
                )

            z = jnp.zeros((16,), jnp.float32)
            accs = lax.fori_loop(0, _SUBLEN, sbody, (z, z, z, z), unroll=5)
            for d in range(4):
                out_v[b, pl.ds(d * 16, 16)] = accs[d] * r
        base = pl.multiple_of(row0 + chunk * _CHUNK, _CHUNK)
        pltpu.sync_copy(out_v, out_hbm.at[pl.ds(base, _CHUNK)])

    issue_idx(0, 0)
    wait_idx(0, 0)
    rcp0 = count_rcp(0)
    issue_gathers(0)
    issue_idx(1, 1)

    def body2(t, rcp_cur):
        for q in range(2):
            i = 2 * t + q
            wait_gathers(q)
            wait_idx(i + 1, 1 - q)
            rcp_next = count_rcp(1 - q)
            issue_gathers(1 - q)
            issue_idx(i + 2, q)
            compute(i, q, rcp_cur)
            rcp_cur = rcp_next
        return rcp_cur

    rcp_cur = lax.fori_loop(0, (_NCHUNKS - 2) // 2, body2, rcp0)

    wait_gathers(0)
    wait_idx(_NCHUNKS - 1, 1)
    rcp_last = count_rcp(1)
    issue_gathers(1)
    compute(_NCHUNKS - 2, 0, rcp_cur)
    wait_gathers(1)
    compute(_NCHUNKS - 1, 1, rcp_last)


@jax.jit
def _sub_token_embed(idx2d, table):
    mesh = plsc.VectorSubcoreMesh(core_axis_name="c", subcore_axis_name="s")
    return pl.kernel(
        _sc_body,
        out_type=jax.ShapeDtypeStruct((_BATCH, _EMBED), jnp.float32),
        mesh=mesh,
        scratch_types=[
            pltpu.VMEM((_CHUNK // 2, _IDX_COLS), jnp.int32),
            pltpu.VMEM((_CHUNK // 2, _IDX_COLS), jnp.int32),
            pltpu.VMEM((_CHUNK * _SUBLEN, _EMBED), jnp.float32),
            pltpu.VMEM((_CHUNK * _SUBLEN, _EMBED), jnp.float32),
            pltpu.VMEM((_CHUNK, _EMBED), jnp.float32),
            pltpu.SemaphoreType.DMA,
            pltpu.SemaphoreType.DMA,
        ],
        compiler_params=pltpu.CompilerParams(
            use_tc_tiling_on_sc=False, needs_layout_passes=False),
    )(idx2d, table)


def kernel(sub_tokens_indices, embeddings_weight):
    idx2d = sub_tokens_indices.astype(jnp.int32).reshape(
        _BATCH // 2, 2 * _SUBLEN)
    return _sub_token_embed(idx2d, embeddings_weight)
